# Initial kernel scaffold; baseline (speedup 1.0000x reference)
import jax
import jax.numpy as jnp
from jax.experimental import pallas as pl


def kernel(nh, eh, edge_index, Wn1, bn1, Wn2, bn2, We1, be1, We2, be2):
    raise NotImplementedError("write your pallas kernel here")


# speedup vs baseline: 1.0651x; 1.0651x over previous
import functools

import jax
import jax.numpy as jnp
from jax import lax
from jax.experimental import pallas as pl
from jax.experimental.pallas import tpu as pltpu

D_IN = 256
D_H = 512
D_OUT = 256


def _mlp_body(x_ref, w1_ref, b1_ref, w2_ref, b2_ref, o_ref):
    h = jnp.dot(x_ref[...], w1_ref[...], preferred_element_type=jnp.float32)
    h = jnp.maximum(h + b1_ref[...], 0.0)
    o = jnp.dot(h, w2_ref[...], preferred_element_type=jnp.float32)
    o_ref[...] = o + b2_ref[...]


def _mlp(x, W1, b1, W2, b2, block_m):
    m = x.shape[0]
    grid = (pl.cdiv(m, block_m),)
    return pl.pallas_call(
        _mlp_body,
        grid=grid,
        in_specs=[
            pl.BlockSpec((block_m, D_IN), lambda i: (i, 0)),
            pl.BlockSpec((D_IN, D_H), lambda i: (0, 0)),
            pl.BlockSpec((1, D_H), lambda i: (0, 0)),
            pl.BlockSpec((D_H, D_OUT), lambda i: (0, 0)),
            pl.BlockSpec((1, D_OUT), lambda i: (0, 0)),
        ],
        out_specs=pl.BlockSpec((block_m, D_OUT), lambda i: (i, 0)),
        out_shape=jax.ShapeDtypeStruct((m, D_OUT), jnp.float32),
    )(x, W1, b1.reshape(1, -1), W2, b2.reshape(1, -1))


def kernel(nh, eh, edge_index, Wn1, bn1, Wn2, bn2, We1, be1, We2, be2):
    N = nh.shape[0]
    src = edge_index[0]
    dst = edge_index[1]

    n_h = _mlp(nh, Wn1, bn1, Wn2, bn2, block_m=1000)
    e_h = _mlp(eh, We1, be1, We2, be2, block_m=2000)

    src_nh = n_h[src]
    dst_nh = n_h[dst]
    msg = src_nh + e_h
    attn = jnp.sum(msg * dst_nh, axis=-1)
    m = jax.ops.segment_max(attn, dst, num_segments=N)
    m = jnp.where(jnp.isfinite(m), m, 0.0)
    ex = jnp.exp(attn - m[dst])
    s = jax.ops.segment_sum(ex, dst, num_segments=N)
    attn_sm = ex / s[dst]
    nz = jax.ops.segment_sum(attn_sm[:, None] * src_nh, dst, num_segments=N)
    n_out = n_h + nz
    e_out = e_h * (1.0 + nz[src] - nz[dst])
    return (n_out, e_out)


# speedup vs baseline: 1.3710x; 1.2872x over previous
import functools

import jax
import jax.numpy as jnp
from jax import lax
from jax.experimental import pallas as pl
from jax.experimental.pallas import tpu as pltpu
from jax.experimental.pallas import tpu_sc as plsc

D_IN = 256
D_H = 512
D_OUT = 256

_NC = 2
_NS = 16
_NW = _NC * _NS


def _gather_rows(table, idx, chunk=200):
    B = idx.shape[0]
    D = table.shape[1]
    b_per_w = B // _NW
    n_chunks = b_per_w // chunk
    mesh = plsc.VectorSubcoreMesh(core_axis_name="c", subcore_axis_name="s")

    @functools.partial(
        pl.kernel,
        mesh=mesh,
        out_type=jax.ShapeDtypeStruct((B, D), jnp.float32),
        scratch_types=[
            pltpu.VMEM((chunk,), jnp.int32),
            pltpu.VMEM((chunk, D), jnp.float32),
            pltpu.SemaphoreType.DMA,
        ],
    )
    def k(table_hbm, idx_hbm, out_hbm, idx_v, rows_v, sem):
        wid = lax.axis_index("s") * _NC + lax.axis_index("c")
        base = wid * b_per_w

        def body(j, carry):
            off = base + j * chunk
            pltpu.sync_copy(idx_hbm.at[pl.ds(off, chunk)], idx_v)
            pltpu.async_copy(table_hbm.at[idx_v], rows_v, sem).wait()
            pltpu.sync_copy(rows_v, out_hbm.at[pl.ds(off, chunk)])
            return carry

        lax.fori_loop(0, n_chunks, body, 0)

    return k(table, idx)


def _mlp_body(x_ref, w1_ref, b1_ref, w2_ref, b2_ref, o_ref):
    h = jnp.dot(x_ref[...], w1_ref[...], preferred_element_type=jnp.float32)
    h = jnp.maximum(h + b1_ref[...], 0.0)
    o = jnp.dot(h, w2_ref[...], preferred_element_type=jnp.float32)
    o_ref[...] = o + b2_ref[...]


def _mlp(x, W1, b1, W2, b2, block_m):
    m = x.shape[0]
    grid = (pl.cdiv(m, block_m),)
    return pl.pallas_call(
        _mlp_body,
        grid=grid,
        in_specs=[
            pl.BlockSpec((block_m, D_IN), lambda i: (i, 0)),
            pl.BlockSpec((D_IN, D_H), lambda i: (0, 0)),
            pl.BlockSpec((1, D_H), lambda i: (0, 0)),
            pl.BlockSpec((D_H, D_OUT), lambda i: (0, 0)),
            pl.BlockSpec((1, D_OUT), lambda i: (0, 0)),
        ],
        out_specs=pl.BlockSpec((block_m, D_OUT), lambda i: (i, 0)),
        out_shape=jax.ShapeDtypeStruct((m, D_OUT), jnp.float32),
    )(x, W1, b1.reshape(1, -1), W2, b2.reshape(1, -1))


def kernel(nh, eh, edge_index, Wn1, bn1, Wn2, bn2, We1, be1, We2, be2):
    N = nh.shape[0]
    src = edge_index[0]
    dst = edge_index[1]

    n_h = _mlp(nh, Wn1, bn1, Wn2, bn2, block_m=1000)
    e_h = _mlp(eh, We1, be1, We2, be2, block_m=2000)

    src_nh = _gather_rows(n_h, src)
    dst_nh = _gather_rows(n_h, dst)
    msg = src_nh + e_h
    attn = jnp.sum(msg * dst_nh, axis=-1)
    m = jax.ops.segment_max(attn, dst, num_segments=N)
    m = jnp.where(jnp.isfinite(m), m, 0.0)
    ex = jnp.exp(attn - m[dst])
    s = jax.ops.segment_sum(ex, dst, num_segments=N)
    attn_sm = ex / s[dst]
    nz = jax.ops.segment_sum(attn_sm[:, None] * src_nh, dst, num_segments=N)
    n_out = n_h + nz
    nz_src = _gather_rows(nz, src)
    nz_dst = _gather_rows(nz, dst)
    e_out = e_h * (1.0 + nz_src - nz_dst)
    return (n_out, e_out)


# speedup vs baseline: 3.0289x; 2.2093x over previous
import functools

import jax
import jax.numpy as jnp
from jax import lax
from jax.experimental import pallas as pl
from jax.experimental.pallas import tpu as pltpu
from jax.experimental.pallas import tpu_sc as plsc

D_IN = 256
D_H = 512
D_OUT = 256

_NC = 2
_NS = 16
_NW = _NC * _NS


def _multi_gather(pairs, chunk=200):
    n_p = len(pairs)
    B = pairs[0][1].shape[0]
    b_per_w = B // _NW
    n_chunks = b_per_w // chunk
    mesh = plsc.VectorSubcoreMesh(core_axis_name="c", subcore_axis_name="s")

    out_type = tuple(
        jax.ShapeDtypeStruct((B, t.shape[1]), jnp.float32) for t, _ in pairs)
    scratch = []
    for t, _ in pairs:
        scratch.append(pltpu.VMEM((chunk,), jnp.int32))
        scratch.append(pltpu.VMEM((chunk, t.shape[1]), jnp.float32))
        scratch.append(pltpu.SemaphoreType.DMA)

    @functools.partial(
        pl.kernel,
        mesh=mesh,
        compiler_params=pltpu.CompilerParams(needs_layout_passes=False),
        out_type=out_type,
        scratch_types=scratch,
    )
    def k(*refs):
        tables = refs[0:n_p]
        idxs = refs[n_p:2 * n_p]
        outs = refs[2 * n_p:3 * n_p]
        scr = refs[3 * n_p:]
        wid = lax.axis_index("s") * _NC + lax.axis_index("c")
        base = wid * b_per_w

        def body(j, carry):
            off = base + j * chunk
            waits = []
            for p in range(n_p):
                iv, rv, sem = scr[3 * p], scr[3 * p + 1], scr[3 * p + 2]
                pltpu.sync_copy(idxs[p].at[pl.ds(off, chunk)], iv)
                waits.append(pltpu.async_copy(tables[p].at[iv], rv, sem))
            for p in range(n_p):
                waits[p].wait()
                pltpu.sync_copy(scr[3 * p + 1], outs[p].at[pl.ds(off, chunk)])
            return carry

        lax.fori_loop(0, n_chunks, body, 0)

    args = [t for t, _ in pairs] + [i for _, i in pairs]
    return k(*args)


def _attn_body(s_ref, d_ref, e_ref, o_ref):
    o_ref[...] = jnp.sum((s_ref[...] + e_ref[...]) * d_ref[...],
                         axis=-1, keepdims=True)


def _attn_rows(src_nh, dst_nh, e_h, block_m=2000):
    n_rows = src_nh.shape[0]
    return pl.pallas_call(
        _attn_body,
        grid=(n_rows // block_m,),
        in_specs=[pl.BlockSpec((block_m, D_OUT), lambda i: (i, 0))] * 3,
        out_specs=pl.BlockSpec((block_m, 1), lambda i: (i, 0)),
        out_shape=jax.ShapeDtypeStruct((n_rows, 1), jnp.float32),
    )(src_nh, dst_nh, e_h)


def _scale_body(a_ref, x_ref, o_ref):
    o_ref[...] = a_ref[...] * x_ref[...]


def _scale_rows(attn_sm, src_nh, block_m=2000):
    n_rows = src_nh.shape[0]
    return pl.pallas_call(
        _scale_body,
        grid=(n_rows // block_m,),
        in_specs=[
            pl.BlockSpec((block_m, 1), lambda i: (i, 0)),
            pl.BlockSpec((block_m, D_OUT), lambda i: (i, 0)),
        ],
        out_specs=pl.BlockSpec((block_m, D_OUT), lambda i: (i, 0)),
        out_shape=jax.ShapeDtypeStruct((n_rows, D_OUT), jnp.float32),
    )(attn_sm.reshape(-1, 1), src_nh)


def _nout_body(n_ref, l_ref, r_ref, o_ref):
    o_ref[:, :128] = n_ref[:, :128] + l_ref[...]
    o_ref[:, 128:] = n_ref[:, 128:] + r_ref[...]


def _node_out(n_h, nzl, nzr, block_m=2000):
    n_rows = n_h.shape[0]
    return pl.pallas_call(
        _nout_body,
        grid=(n_rows // block_m,),
        in_specs=[
            pl.BlockSpec((block_m, D_OUT), lambda i: (i, 0)),
            pl.BlockSpec((block_m, 128), lambda i: (i, 0)),
            pl.BlockSpec((block_m, 128), lambda i: (i, 0)),
        ],
        out_specs=pl.BlockSpec((block_m, D_OUT), lambda i: (i, 0)),
        out_shape=jax.ShapeDtypeStruct((n_rows, D_OUT), jnp.float32),
    )(n_h, nzl, nzr)


def _eout_body(e_ref, sl_ref, sr_ref, dl_ref, dr_ref, o_ref):
    o_ref[:, :128] = e_ref[:, :128] * (1.0 + sl_ref[...] - dl_ref[...])
    o_ref[:, 128:] = e_ref[:, 128:] * (1.0 + sr_ref[...] - dr_ref[...])


def _edge_out(e_h, nsl, nsr, ndl, ndr, block_m=2000):
    n_rows = e_h.shape[0]
    half = pl.BlockSpec((block_m, 128), lambda i: (i, 0))
    return pl.pallas_call(
        _eout_body,
        grid=(n_rows // block_m,),
        in_specs=[pl.BlockSpec((block_m, D_OUT), lambda i: (i, 0)),
                  half, half, half, half],
        out_specs=pl.BlockSpec((block_m, D_OUT), lambda i: (i, 0)),
        out_shape=jax.ShapeDtypeStruct((n_rows, D_OUT), jnp.float32),
    )(e_h, nsl, nsr, ndl, ndr)


def _mlp_body(x_ref, w1_ref, b1_ref, w2_ref, b2_ref, o_ref):
    h = jnp.dot(x_ref[...], w1_ref[...], preferred_element_type=jnp.float32)
    h = jnp.maximum(h + b1_ref[...], 0.0)
    o = jnp.dot(h, w2_ref[...], preferred_element_type=jnp.float32)
    o_ref[...] = o + b2_ref[...]


def _mlp(x, W1, b1, W2, b2, block_m):
    m = x.shape[0]
    grid = (pl.cdiv(m, block_m),)
    return pl.pallas_call(
        _mlp_body,
        grid=grid,
        in_specs=[
            pl.BlockSpec((block_m, D_IN), lambda i: (i, 0)),
            pl.BlockSpec((D_IN, D_H), lambda i: (0, 0)),
            pl.BlockSpec((1, D_H), lambda i: (0, 0)),
            pl.BlockSpec((D_H, D_OUT), lambda i: (0, 0)),
            pl.BlockSpec((1, D_OUT), lambda i: (0, 0)),
        ],
        out_specs=pl.BlockSpec((block_m, D_OUT), lambda i: (i, 0)),
        out_shape=jax.ShapeDtypeStruct((m, D_OUT), jnp.float32),
    )(x, W1, b1.reshape(1, -1), W2, b2.reshape(1, -1))


_L = 16
_NP = 10240
_EP = 163840
_CH = 128


def _seg_stats(attn_p, dst_p):
    b_per_w = _EP // _NW
    mesh = plsc.VectorSubcoreMesh(core_axis_name="c", subcore_axis_name="s")

    @functools.partial(
        pl.kernel,
        mesh=mesh,
        compiler_params=pltpu.CompilerParams(needs_layout_passes=False),
        out_type=(
            jax.ShapeDtypeStruct((_NW, _NP), jnp.float32),
            jax.ShapeDtypeStruct((_NW, _NP), jnp.float32),
        ),
        scratch_types=[
            pltpu.VMEM((b_per_w,), jnp.float32),
            pltpu.VMEM((b_per_w,), jnp.int32),
            pltpu.VMEM((_NP,), jnp.float32),
            pltpu.VMEM((_NP,), jnp.float32),
        ],
    )
    def k(attn_hbm, dst_hbm, m_hbm, s_hbm, a_v, d_v, m_v, s_v):
        wid = lax.axis_index("s") * _NC + lax.axis_index("c")
        base = wid * b_per_w
        pltpu.sync_copy(attn_hbm.at[pl.ds(base, b_per_w)], a_v)
        pltpu.sync_copy(dst_hbm.at[pl.ds(base, b_per_w)], d_v)

        neg = jnp.full((_L,), -1e30, jnp.float32)
        zero = jnp.zeros((_L,), jnp.float32)

        def init_body(i, c):
            m_v[pl.ds(i * _L, _L)] = neg
            s_v[pl.ds(i * _L, _L)] = zero
            return c
        lax.fori_loop(0, _NP // _L, init_body, 0)

        def max_body(i, c):
            d = d_v[pl.ds(i * _L, _L)]
            a = a_v[pl.ds(i * _L, _L)]
            cur = plsc.load_gather(m_v, [d])
            plsc.store_scatter(m_v, [d], jnp.maximum(cur, a))
            return c
        lax.fori_loop(0, b_per_w // _L, max_body, 0)

        def sum_body(i, c):
            d = d_v[pl.ds(i * _L, _L)]
            a = a_v[pl.ds(i * _L, _L)]
            mv = plsc.load_gather(m_v, [d])
            plsc.addupdate_scatter(s_v, [d], jnp.exp(a - mv))
            return c
        lax.fori_loop(0, b_per_w // _L, sum_body, 0)

        pltpu.sync_copy(m_v, m_hbm.at[wid])
        pltpu.sync_copy(s_v, s_hbm.at[wid])

    return k(attn_p, dst_p)


def _merge_body(m_ref, s_ref, mg_ref, sg_ref):
    m = m_ref[...]
    s = s_ref[...]
    mg = jnp.max(m, axis=0, keepdims=True)
    scale = jnp.where(s > 0.0, jnp.exp(m - mg), 0.0)
    sg_ref[...] = jnp.sum(s * scale, axis=0, keepdims=True)
    mg_ref[...] = mg


def _merge_stats(m_parts, s_parts, block=2048):
    grid = (_NP // block,)
    return pl.pallas_call(
        _merge_body,
        grid=grid,
        in_specs=[
            pl.BlockSpec((_NW, block), lambda i: (0, i)),
            pl.BlockSpec((_NW, block), lambda i: (0, i)),
        ],
        out_specs=(
            pl.BlockSpec((1, block), lambda i: (0, i)),
            pl.BlockSpec((1, block), lambda i: (0, i)),
        ),
        out_shape=(
            jax.ShapeDtypeStruct((1, _NP), jnp.float32),
            jax.ShapeDtypeStruct((1, _NP), jnp.float32),
        ),
    )(m_parts, s_parts)


def _edge_weights(attn_p, dst_p, m_g, s_g):
    b_per_w = _EP // _NW
    mesh = plsc.VectorSubcoreMesh(core_axis_name="c", subcore_axis_name="s")

    @functools.partial(
        pl.kernel,
        mesh=mesh,
        compiler_params=pltpu.CompilerParams(needs_layout_passes=False),
        out_type=jax.ShapeDtypeStruct((_EP,), jnp.float32),
        scratch_types=[
            pltpu.VMEM((b_per_w,), jnp.float32),
            pltpu.VMEM((b_per_w,), jnp.int32),
            pltpu.VMEM((_NP,), jnp.float32),
            pltpu.VMEM((_NP,), jnp.float32),
        ],
    )
    def k(attn_hbm, dst_hbm, mg_hbm, sg_hbm, out_hbm, a_v, d_v, mg_v, sg_v):
        wid = lax.axis_index("s") * _NC + lax.axis_index("c")
        base = wid * b_per_w
        pltpu.sync_copy(attn_hbm.at[pl.ds(base, b_per_w)], a_v)
        pltpu.sync_copy(dst_hbm.at[pl.ds(base, b_per_w)], d_v)
        pltpu.sync_copy(mg_hbm.at[pl.ds(0, _NP)], mg_v)
        pltpu.sync_copy(sg_hbm.at[pl.ds(0, _NP)], sg_v)

        def w_body(i, c):
            d = d_v[pl.ds(i * _L, _L)]
            a = a_v[pl.ds(i * _L, _L)]
            mv = plsc.load_gather(mg_v, [d])
            sv = plsc.load_gather(sg_v, [d])
            a_v[pl.ds(i * _L, _L)] = jnp.exp(a - mv) / sv
            return c
        lax.fori_loop(0, b_per_w // _L, w_body, 0)

        pltpu.sync_copy(a_v, out_hbm.at[pl.ds(base, b_per_w)])

    return k(attn_p, dst_p, m_g, s_g)


def _scatter_rows(scaled, dst3, n_edges):
    e_per_tile = _EP // _NS
    n_chunks_full = e_per_tile // _CH
    rows_per_tile = _NP // _NS
    Dh = 128
    mesh = plsc.VectorSubcoreMesh(core_axis_name="c", subcore_axis_name="s")

    @functools.partial(
        pl.kernel,
        mesh=mesh,
        compiler_params=pltpu.CompilerParams(needs_layout_passes=False),
        out_type=jax.ShapeDtypeStruct((2, _NP, Dh), jnp.float32),
        scratch_types=[
            pltpu.VMEM((_CH,), jnp.int32),
            pltpu.VMEM((_CH, Dh), jnp.float32),
            pltpu.VMEM_SHARED((_NP, Dh), jnp.float32),
        ],
    )
    def k(scaled_hbm, dst3_hbm, nz_hbm, idx_v, rows_v, nz_sh):
        cid = lax.axis_index("c")
        sid = lax.axis_index("s")
        base = sid * e_per_tile

        zero = jnp.zeros((_L,), jnp.float32)

        def zr_body(r, c):
            for kk in range(Dh // _L):
                rows_v[r, pl.ds(kk * _L, _L)] = zero
            return c
        lax.fori_loop(0, _CH, zr_body, 0)
        for part in range(rows_per_tile // _CH):
            pltpu.sync_copy(
                rows_v,
                nz_sh.at[pl.ds(sid * rows_per_tile + part * _CH, _CH)])
        rem_rows = rows_per_tile % _CH
        if rem_rows:
            pltpu.sync_copy(
                rows_v.at[pl.ds(0, rem_rows)],
                nz_sh.at[pl.ds(sid * rows_per_tile
                               + (rows_per_tile // _CH) * _CH, rem_rows)])
        plsc.subcore_barrier()

        def chunk_body(j, c):
            off = base + j * _CH
            pltpu.sync_copy(scaled_hbm.at[cid, pl.ds(off, _CH)], rows_v)
            pltpu.sync_copy(dst3_hbm.at[sid, j], idx_v)
            pltpu.sync_copy(rows_v, nz_sh.at[idx_v], add=True)
            return c
        lax.fori_loop(0, n_chunks_full, chunk_body, 0)

        plsc.subcore_barrier()
        pltpu.sync_copy(
            nz_sh.at[pl.ds(sid * rows_per_tile, rows_per_tile)],
            nz_hbm.at[cid, pl.ds(sid * rows_per_tile, rows_per_tile)])

    return k(scaled, dst3)


def kernel(nh, eh, edge_index, Wn1, bn1, Wn2, bn2, We1, be1, We2, be2):
    N = nh.shape[0]
    src = edge_index[0]
    dst = edge_index[1]

    n_h = _mlp(nh, Wn1, bn1, Wn2, bn2, block_m=1000)
    e_h = _mlp(eh, We1, be1, We2, be2, block_m=2000)

    E = src.shape[0]
    src_nh, dst_nh = _multi_gather([(n_h, src), (n_h, dst)])
    attn = _attn_rows(src_nh, dst_nh, e_h).reshape(E)

    pad = _EP - E
    attn_p = jnp.concatenate([attn, jnp.full((pad,), -1e30, jnp.float32)])
    dst_p = jnp.concatenate([dst, jnp.full((pad,), N, jnp.int32)])
    dst3 = dst_p.reshape(_NS, (_EP // _NS) // _CH, _CH)

    _BISECT_XLA_SOFTMAX = False
    if _BISECT_XLA_SOFTMAX:
        m = jax.ops.segment_max(attn, dst, num_segments=N)
        m = jnp.where(jnp.isfinite(m), m, 0.0)
        ex = jnp.exp(attn - m[dst])
        s = jax.ops.segment_sum(ex, dst, num_segments=N)
        attn_sm = ex / s[dst]
    else:
        m_parts, s_parts = _seg_stats(attn_p, dst_p)
        m_g, s_g = _merge_stats(m_parts, s_parts)
        attn_sm = _edge_weights(attn_p, dst_p,
                                m_g.reshape(-1), s_g.reshape(-1))[:E]

    scaled = _scale_rows(attn_sm, src_nh)
    scaled_p = jnp.concatenate(
        [scaled, jnp.zeros((_EP - E, D_OUT), jnp.float32)])
    scaled2 = jnp.stack([scaled_p[:, :128], scaled_p[:, 128:]])
    nz3 = _scatter_rows(scaled2, dst3, E)
    nzl, nzr = nz3[0], nz3[1]
    n_out = _node_out(n_h, nzl[:N], nzr[:N])
    nsl, nsr, ndl, ndr = _multi_gather(
        [(nzl, src), (nzr, src), (nzl, dst), (nzr, dst)])
    e_out = _edge_out(e_h, nsl, nsr, ndl, ndr)
    return (n_out, e_out)


# speedup vs baseline: 3.2224x; 1.0639x over previous
import functools

import jax
import jax.numpy as jnp
from jax import lax
from jax.experimental import pallas as pl
from jax.experimental.pallas import tpu as pltpu
from jax.experimental.pallas import tpu_sc as plsc

D_IN = 256
D_H = 512
D_OUT = 256

_NC = 2
_NS = 16
_NW = _NC * _NS


def _multi_gather(pairs, chunk=200):
    n_p = len(pairs)
    B = pairs[0][1].shape[0]
    b_per_w = B // _NW
    n_chunks = b_per_w // chunk
    mesh = plsc.VectorSubcoreMesh(core_axis_name="c", subcore_axis_name="s")

    out_type = tuple(
        jax.ShapeDtypeStruct((B, t.shape[1]), jnp.float32) for t, _ in pairs)
    scratch = []
    for t, _ in pairs:
        scratch.append(pltpu.VMEM((chunk,), jnp.int32))
        scratch.append(pltpu.VMEM((chunk, t.shape[1]), jnp.float32))
        scratch.append(pltpu.SemaphoreType.DMA)

    @functools.partial(
        pl.kernel,
        mesh=mesh,
        compiler_params=pltpu.CompilerParams(needs_layout_passes=False),
        out_type=out_type,
        scratch_types=scratch,
    )
    def k(*refs):
        tables = refs[0:n_p]
        idxs = refs[n_p:2 * n_p]
        outs = refs[2 * n_p:3 * n_p]
        scr = refs[3 * n_p:]
        wid = lax.axis_index("s") * _NC + lax.axis_index("c")
        base = wid * b_per_w

        def body(j, carry):
            off = base + j * chunk
            waits = []
            for p in range(n_p):
                iv, rv, sem = scr[3 * p], scr[3 * p + 1], scr[3 * p + 2]
                pltpu.sync_copy(idxs[p].at[pl.ds(off, chunk)], iv)
                waits.append(pltpu.async_copy(tables[p].at[iv], rv, sem))
            for p in range(n_p):
                waits[p].wait()
                pltpu.sync_copy(scr[3 * p + 1], outs[p].at[pl.ds(off, chunk)])
            return carry

        lax.fori_loop(0, n_chunks, body, 0)

    args = [t for t, _ in pairs] + [i for _, i in pairs]
    return k(*args)


def _attn_body(s_ref, d_ref, e_ref, o_ref):
    o_ref[...] = jnp.sum((s_ref[...] + e_ref[...]) * d_ref[...],
                         axis=-1, keepdims=True)


def _attn_rows(src_nh, dst_nh, e_h, block_m=2000):
    n_rows = src_nh.shape[0]
    return pl.pallas_call(
        _attn_body,
        grid=(n_rows // block_m,),
        in_specs=[pl.BlockSpec((block_m, D_OUT), lambda i: (i, 0))] * 3,
        out_specs=pl.BlockSpec((block_m, 1), lambda i: (i, 0)),
        out_shape=jax.ShapeDtypeStruct((n_rows, 1), jnp.float32),
    )(src_nh, dst_nh, e_h)


def _scale_body(a_ref, x_ref, o_ref):
    o_ref[0] = a_ref[...] * x_ref[:, :128]
    o_ref[1] = a_ref[...] * x_ref[:, 128:]


def _scale_rows_split(attn_sm, src_nh, block_m=2000):
    n_rows = src_nh.shape[0]
    return pl.pallas_call(
        _scale_body,
        grid=(n_rows // block_m,),
        in_specs=[
            pl.BlockSpec((block_m, 1), lambda i: (i, 0)),
            pl.BlockSpec((block_m, D_OUT), lambda i: (i, 0)),
        ],
        out_specs=pl.BlockSpec((2, block_m, 128), lambda i: (0, i, 0)),
        out_shape=jax.ShapeDtypeStruct((2, n_rows, 128), jnp.float32),
    )(attn_sm.reshape(-1, 1), src_nh)


def _nout_body(n_ref, l_ref, r_ref, o_ref):
    o_ref[:, :128] = n_ref[:, :128] + l_ref[...]
    o_ref[:, 128:] = n_ref[:, 128:] + r_ref[...]


def _node_out(n_h, nzl, nzr, block_m=2000):
    n_rows = n_h.shape[0]
    return pl.pallas_call(
        _nout_body,
        grid=(n_rows // block_m,),
        in_specs=[
            pl.BlockSpec((block_m, D_OUT), lambda i: (i, 0)),
            pl.BlockSpec((block_m, 128), lambda i: (i, 0)),
            pl.BlockSpec((block_m, 128), lambda i: (i, 0)),
        ],
        out_specs=pl.BlockSpec((block_m, D_OUT), lambda i: (i, 0)),
        out_shape=jax.ShapeDtypeStruct((n_rows, D_OUT), jnp.float32),
    )(n_h, nzl, nzr)


def _eout_body(e_ref, sl_ref, sr_ref, dl_ref, dr_ref, o_ref):
    o_ref[:, :128] = e_ref[:, :128] * (1.0 + sl_ref[...] - dl_ref[...])
    o_ref[:, 128:] = e_ref[:, 128:] * (1.0 + sr_ref[...] - dr_ref[...])


def _edge_out(e_h, nsl, nsr, ndl, ndr, block_m=2000):
    n_rows = e_h.shape[0]
    half = pl.BlockSpec((block_m, 128), lambda i: (i, 0))
    return pl.pallas_call(
        _eout_body,
        grid=(n_rows // block_m,),
        in_specs=[pl.BlockSpec((block_m, D_OUT), lambda i: (i, 0)),
                  half, half, half, half],
        out_specs=pl.BlockSpec((block_m, D_OUT), lambda i: (i, 0)),
        out_shape=jax.ShapeDtypeStruct((n_rows, D_OUT), jnp.float32),
    )(e_h, nsl, nsr, ndl, ndr)


def _mlp_body(x_ref, w1_ref, b1_ref, w2_ref, b2_ref, o_ref):
    h = jnp.dot(x_ref[...], w1_ref[...], preferred_element_type=jnp.float32)
    h = jnp.maximum(h + b1_ref[...], 0.0)
    o = jnp.dot(h, w2_ref[...], preferred_element_type=jnp.float32)
    o_ref[...] = o + b2_ref[...]


def _mlp(x, W1, b1, W2, b2, block_m):
    m = x.shape[0]
    grid = (pl.cdiv(m, block_m),)
    return pl.pallas_call(
        _mlp_body,
        grid=grid,
        in_specs=[
            pl.BlockSpec((block_m, D_IN), lambda i: (i, 0)),
            pl.BlockSpec((D_IN, D_H), lambda i: (0, 0)),
            pl.BlockSpec((1, D_H), lambda i: (0, 0)),
            pl.BlockSpec((D_H, D_OUT), lambda i: (0, 0)),
            pl.BlockSpec((1, D_OUT), lambda i: (0, 0)),
        ],
        out_specs=pl.BlockSpec((block_m, D_OUT), lambda i: (i, 0)),
        out_shape=jax.ShapeDtypeStruct((m, D_OUT), jnp.float32),
    )(x, W1, b1.reshape(1, -1), W2, b2.reshape(1, -1))


_L = 16
_NP = 10240
_EP = 163840
_CH = 128


def _seg_stats(attn_p, dst_p):
    b_per_w = _EP // _NW
    mesh = plsc.VectorSubcoreMesh(core_axis_name="c", subcore_axis_name="s")

    @functools.partial(
        pl.kernel,
        mesh=mesh,
        compiler_params=pltpu.CompilerParams(needs_layout_passes=False),
        out_type=(
            jax.ShapeDtypeStruct((_NW, _NP), jnp.float32),
            jax.ShapeDtypeStruct((_NW, _NP), jnp.float32),
        ),
        scratch_types=[
            pltpu.VMEM((b_per_w,), jnp.float32),
            pltpu.VMEM((b_per_w,), jnp.int32),
            pltpu.VMEM((_NP,), jnp.float32),
            pltpu.VMEM((_NP,), jnp.float32),
        ],
    )
    def k(attn_hbm, dst_hbm, m_hbm, s_hbm, a_v, d_v, m_v, s_v):
        wid = lax.axis_index("s") * _NC + lax.axis_index("c")
        base = wid * b_per_w
        pltpu.sync_copy(attn_hbm.at[pl.ds(base, b_per_w)], a_v)
        pltpu.sync_copy(dst_hbm.at[pl.ds(base, b_per_w)], d_v)

        neg = jnp.full((_L,), -1e30, jnp.float32)
        zero = jnp.zeros((_L,), jnp.float32)

        def init_body(i, c):
            m_v[pl.ds(i * _L, _L)] = neg
            s_v[pl.ds(i * _L, _L)] = zero
            return c
        lax.fori_loop(0, _NP // _L, init_body, 0)

        def max_body(i, c):
            d = d_v[pl.ds(i * _L, _L)]
            a = a_v[pl.ds(i * _L, _L)]
            cur = plsc.load_gather(m_v, [d])
            plsc.store_scatter(m_v, [d], jnp.maximum(cur, a))
            return c
        lax.fori_loop(0, b_per_w // _L, max_body, 0)

        def sum_body(i, c):
            d = d_v[pl.ds(i * _L, _L)]
            a = a_v[pl.ds(i * _L, _L)]
            mv = plsc.load_gather(m_v, [d])
            plsc.addupdate_scatter(s_v, [d], jnp.exp(a - mv))
            return c
        lax.fori_loop(0, b_per_w // _L, sum_body, 0)

        pltpu.sync_copy(m_v, m_hbm.at[wid])
        pltpu.sync_copy(s_v, s_hbm.at[wid])

    return k(attn_p, dst_p)


def _merge_body(m_ref, s_ref, mg_ref, sg_ref):
    m = m_ref[...]
    s = s_ref[...]
    mg = jnp.max(m, axis=0, keepdims=True)
    scale = jnp.where(s > 0.0, jnp.exp(m - mg), 0.0)
    sg_ref[...] = jnp.sum(s * scale, axis=0, keepdims=True)
    mg_ref[...] = mg


def _merge_stats(m_parts, s_parts, block=2048):
    grid = (_NP // block,)
    return pl.pallas_call(
        _merge_body,
        grid=grid,
        in_specs=[
            pl.BlockSpec((_NW, block), lambda i: (0, i)),
            pl.BlockSpec((_NW, block), lambda i: (0, i)),
        ],
        out_specs=(
            pl.BlockSpec((1, block), lambda i: (0, i)),
            pl.BlockSpec((1, block), lambda i: (0, i)),
        ),
        out_shape=(
            jax.ShapeDtypeStruct((1, _NP), jnp.float32),
            jax.ShapeDtypeStruct((1, _NP), jnp.float32),
        ),
    )(m_parts, s_parts)


def _edge_weights(attn_p, dst_p, m_g, s_g):
    b_per_w = _EP // _NW
    mesh = plsc.VectorSubcoreMesh(core_axis_name="c", subcore_axis_name="s")

    @functools.partial(
        pl.kernel,
        mesh=mesh,
        compiler_params=pltpu.CompilerParams(needs_layout_passes=False),
        out_type=jax.ShapeDtypeStruct((_EP,), jnp.float32),
        scratch_types=[
            pltpu.VMEM((b_per_w,), jnp.float32),
            pltpu.VMEM((b_per_w,), jnp.int32),
            pltpu.VMEM((_NP,), jnp.float32),
            pltpu.VMEM((_NP,), jnp.float32),
        ],
    )
    def k(attn_hbm, dst_hbm, mg_hbm, sg_hbm, out_hbm, a_v, d_v, mg_v, sg_v):
        wid = lax.axis_index("s") * _NC + lax.axis_index("c")
        base = wid * b_per_w
        pltpu.sync_copy(attn_hbm.at[pl.ds(base, b_per_w)], a_v)
        pltpu.sync_copy(dst_hbm.at[pl.ds(base, b_per_w)], d_v)
        pltpu.sync_copy(mg_hbm.at[pl.ds(0, _NP)], mg_v)
        pltpu.sync_copy(sg_hbm.at[pl.ds(0, _NP)], sg_v)

        def w_body(i, c):
            d = d_v[pl.ds(i * _L, _L)]
            a = a_v[pl.ds(i * _L, _L)]
            mv = plsc.load_gather(mg_v, [d])
            sv = plsc.load_gather(sg_v, [d])
            a_v[pl.ds(i * _L, _L)] = jnp.exp(a - mv) / sv
            return c
        lax.fori_loop(0, b_per_w // _L, w_body, 0)

        pltpu.sync_copy(a_v, out_hbm.at[pl.ds(base, b_per_w)])

    return k(attn_p, dst_p, m_g, s_g)


def _scatter_rows(scaled, dst3, n_edges):
    e_per_tile = _EP // _NS
    n_chunks_full = e_per_tile // _CH
    rows_per_tile = _NP // _NS
    Dh = 128
    mesh = plsc.VectorSubcoreMesh(core_axis_name="c", subcore_axis_name="s")

    @functools.partial(
        pl.kernel,
        mesh=mesh,
        compiler_params=pltpu.CompilerParams(needs_layout_passes=False),
        out_type=jax.ShapeDtypeStruct((2, _NP, Dh), jnp.float32),
        scratch_types=[
            pltpu.VMEM((_CH,), jnp.int32),
            pltpu.VMEM((_CH, Dh), jnp.float32),
            pltpu.VMEM_SHARED((_NP, Dh), jnp.float32),
        ],
    )
    def k(scaled_hbm, dst3_hbm, nz_hbm, idx_v, rows_v, nz_sh):
        cid = lax.axis_index("c")
        sid = lax.axis_index("s")
        base = sid * e_per_tile

        zero = jnp.zeros((_L,), jnp.float32)

        def zr_body(r, c):
            for kk in range(Dh // _L):
                rows_v[r, pl.ds(kk * _L, _L)] = zero
            return c
        lax.fori_loop(0, _CH, zr_body, 0)
        for part in range(rows_per_tile // _CH):
            pltpu.sync_copy(
                rows_v,
                nz_sh.at[pl.ds(sid * rows_per_tile + part * _CH, _CH)])
        rem_rows = rows_per_tile % _CH
        if rem_rows:
            pltpu.sync_copy(
                rows_v.at[pl.ds(0, rem_rows)],
                nz_sh.at[pl.ds(sid * rows_per_tile
                               + (rows_per_tile // _CH) * _CH, rem_rows)])
        plsc.subcore_barrier()

        def chunk_body(j, c):
            off = base + j * _CH
            pltpu.sync_copy(scaled_hbm.at[cid, pl.ds(off, _CH)], rows_v)
            pltpu.sync_copy(dst3_hbm.at[sid, j], idx_v)
            pltpu.sync_copy(rows_v, nz_sh.at[idx_v], add=True)
            return c
        lax.fori_loop(0, n_chunks_full, chunk_body, 0)

        plsc.subcore_barrier()
        pltpu.sync_copy(
            nz_sh.at[pl.ds(sid * rows_per_tile, rows_per_tile)],
            nz_hbm.at[cid, pl.ds(sid * rows_per_tile, rows_per_tile)])

    return k(scaled, dst3)


def kernel(nh, eh, edge_index, Wn1, bn1, Wn2, bn2, We1, be1, We2, be2):
    N = nh.shape[0]
    src = edge_index[0]
    dst = edge_index[1]

    n_h = _mlp(nh, Wn1, bn1, Wn2, bn2, block_m=1000)
    e_h = _mlp(eh, We1, be1, We2, be2, block_m=2000)

    E = src.shape[0]
    src_nh, dst_nh = _multi_gather([(n_h, src), (n_h, dst)])
    attn = _attn_rows(src_nh, dst_nh, e_h).reshape(E)

    pad = _EP - E
    attn_p = jnp.concatenate([attn, jnp.full((pad,), -1e30, jnp.float32)])
    dst_p = jnp.concatenate([dst, jnp.full((pad,), N, jnp.int32)])
    dst3 = dst_p.reshape(_NS, (_EP // _NS) // _CH, _CH)

    _BISECT_XLA_SOFTMAX = False
    if _BISECT_XLA_SOFTMAX:
        m = jax.ops.segment_max(attn, dst, num_segments=N)
        m = jnp.where(jnp.isfinite(m), m, 0.0)
        ex = jnp.exp(attn - m[dst])
        s = jax.ops.segment_sum(ex, dst, num_segments=N)
        attn_sm = ex / s[dst]
    else:
        m_parts, s_parts = _seg_stats(attn_p, dst_p)
        m_g, s_g = _merge_stats(m_parts, s_parts)
        attn_sm = _edge_weights(attn_p, dst_p,
                                m_g.reshape(-1), s_g.reshape(-1))[:E]

    scaled2 = jnp.concatenate(
        [_scale_rows_split(attn_sm, src_nh),
         jnp.zeros((2, _EP - E, 128), jnp.float32)], axis=1)
    nz3 = _scatter_rows(scaled2, dst3, E)
    nzl, nzr = nz3[0], nz3[1]
    n_out = _node_out(n_h, nzl[:N], nzr[:N])
    nsl, nsr, ndl, ndr = _multi_gather(
        [(nzl, src), (nzr, src), (nzl, dst), (nzr, dst)])
    e_out = _edge_out(e_h, nsl, nsr, ndl, ndr)
    return (n_out, e_out)


# speedup vs baseline: 3.2238x; 1.0004x over previous
import functools

import jax
import jax.numpy as jnp
from jax import lax
from jax.experimental import pallas as pl
from jax.experimental.pallas import tpu as pltpu
from jax.experimental.pallas import tpu_sc as plsc

D_IN = 256
D_H = 512
D_OUT = 256

_NC = 2
_NS = 16
_NW = _NC * _NS


def _multi_gather(pairs, chunk=200):
    n_p = len(pairs)
    B = pairs[0][1].shape[0]
    b_per_w = B // _NW
    n_chunks = b_per_w // chunk
    mesh = plsc.VectorSubcoreMesh(core_axis_name="c", subcore_axis_name="s")

    out_type = tuple(
        jax.ShapeDtypeStruct((B, t.shape[1]), jnp.float32) for t, _ in pairs)
    scratch = []
    for t, _ in pairs:
        scratch.append(pltpu.VMEM((chunk,), jnp.int32))
        scratch.append(pltpu.VMEM((chunk, t.shape[1]), jnp.float32))
        scratch.append(pltpu.SemaphoreType.DMA)

    @functools.partial(
        pl.kernel,
        mesh=mesh,
        compiler_params=pltpu.CompilerParams(needs_layout_passes=False),
        out_type=out_type,
        scratch_types=scratch,
    )
    def k(*refs):
        tables = refs[0:n_p]
        idxs = refs[n_p:2 * n_p]
        outs = refs[2 * n_p:3 * n_p]
        scr = refs[3 * n_p:]
        wid = lax.axis_index("s") * _NC + lax.axis_index("c")
        base = wid * b_per_w

        def body(j, carry):
            off = base + j * chunk
            waits = []
            for p in range(n_p):
                iv, rv, sem = scr[3 * p], scr[3 * p + 1], scr[3 * p + 2]
                pltpu.sync_copy(idxs[p].at[pl.ds(off, chunk)], iv)
                waits.append(pltpu.async_copy(tables[p].at[iv], rv, sem))
            for p in range(n_p):
                waits[p].wait()
                pltpu.sync_copy(scr[3 * p + 1], outs[p].at[pl.ds(off, chunk)])
            return carry

        lax.fori_loop(0, n_chunks, body, 0)

    args = [t for t, _ in pairs] + [i for _, i in pairs]
    return k(*args)


def _attn_body(s_ref, d_ref, e_ref, o_ref):
    o_ref[...] = jnp.sum((s_ref[...] + e_ref[...]) * d_ref[...],
                         axis=-1, keepdims=True)


def _attn_rows(src_nh, dst_nh, e_h, block_m=2000):
    n_rows = src_nh.shape[0]
    return pl.pallas_call(
        _attn_body,
        grid=(n_rows // block_m,),
        in_specs=[pl.BlockSpec((block_m, D_OUT), lambda i: (i, 0))] * 3,
        out_specs=pl.BlockSpec((block_m, 1), lambda i: (i, 0)),
        out_shape=jax.ShapeDtypeStruct((n_rows, 1), jnp.float32),
    )(src_nh, dst_nh, e_h)


def _scale_body(a_ref, x_ref, o_ref):
    o_ref[0] = a_ref[...] * x_ref[:, :128]
    o_ref[1] = a_ref[...] * x_ref[:, 128:]


def _scale_rows_split(attn_sm, src_nh, block_m=2000):
    n_rows = src_nh.shape[0]
    return pl.pallas_call(
        _scale_body,
        grid=(n_rows // block_m,),
        in_specs=[
            pl.BlockSpec((block_m, 1), lambda i: (i, 0)),
            pl.BlockSpec((block_m, D_OUT), lambda i: (i, 0)),
        ],
        out_specs=pl.BlockSpec((2, block_m, 128), lambda i: (0, i, 0)),
        out_shape=jax.ShapeDtypeStruct((2, n_rows, 128), jnp.float32),
    )(attn_sm.reshape(-1, 1), src_nh)


def _nout_body(n_ref, l_ref, r_ref, o_ref):
    o_ref[:, :128] = n_ref[:, :128] + l_ref[...]
    o_ref[:, 128:] = n_ref[:, 128:] + r_ref[...]


def _node_out(n_h, nzl, nzr, block_m=2000):
    n_rows = n_h.shape[0]
    return pl.pallas_call(
        _nout_body,
        grid=(n_rows // block_m,),
        in_specs=[
            pl.BlockSpec((block_m, D_OUT), lambda i: (i, 0)),
            pl.BlockSpec((block_m, 128), lambda i: (i, 0)),
            pl.BlockSpec((block_m, 128), lambda i: (i, 0)),
        ],
        out_specs=pl.BlockSpec((block_m, D_OUT), lambda i: (i, 0)),
        out_shape=jax.ShapeDtypeStruct((n_rows, D_OUT), jnp.float32),
    )(n_h, nzl, nzr)


def _eout_body(e_ref, sl_ref, sr_ref, dl_ref, dr_ref, o_ref):
    o_ref[:, :128] = e_ref[:, :128] * (1.0 + sl_ref[...] - dl_ref[...])
    o_ref[:, 128:] = e_ref[:, 128:] * (1.0 + sr_ref[...] - dr_ref[...])


def _edge_out(e_h, nsl, nsr, ndl, ndr, block_m=2000):
    n_rows = e_h.shape[0]
    half = pl.BlockSpec((block_m, 128), lambda i: (i, 0))
    return pl.pallas_call(
        _eout_body,
        grid=(n_rows // block_m,),
        in_specs=[pl.BlockSpec((block_m, D_OUT), lambda i: (i, 0)),
                  half, half, half, half],
        out_specs=pl.BlockSpec((block_m, D_OUT), lambda i: (i, 0)),
        out_shape=jax.ShapeDtypeStruct((n_rows, D_OUT), jnp.float32),
    )(e_h, nsl, nsr, ndl, ndr)


def _mlp_body(x_ref, w1_ref, b1_ref, w2_ref, b2_ref, o_ref):
    h = jnp.dot(x_ref[...], w1_ref[...], preferred_element_type=jnp.float32)
    h = jnp.maximum(h + b1_ref[...], 0.0)
    o = jnp.dot(h, w2_ref[...], preferred_element_type=jnp.float32)
    o_ref[...] = o + b2_ref[...]


def _mlp(x, W1, b1, W2, b2, block_m):
    m = x.shape[0]
    grid = (pl.cdiv(m, block_m),)
    return pl.pallas_call(
        _mlp_body,
        grid=grid,
        in_specs=[
            pl.BlockSpec((block_m, D_IN), lambda i: (i, 0)),
            pl.BlockSpec((D_IN, D_H), lambda i: (0, 0)),
            pl.BlockSpec((1, D_H), lambda i: (0, 0)),
            pl.BlockSpec((D_H, D_OUT), lambda i: (0, 0)),
            pl.BlockSpec((1, D_OUT), lambda i: (0, 0)),
        ],
        out_specs=pl.BlockSpec((block_m, D_OUT), lambda i: (i, 0)),
        out_shape=jax.ShapeDtypeStruct((m, D_OUT), jnp.float32),
    )(x, W1, b1.reshape(1, -1), W2, b2.reshape(1, -1))


_L = 16
_NP = 10240
_EP = 163840
_CH = 128


def _seg_stats(attn_p, dst_p):
    b_per_w = _EP // _NW
    mesh = plsc.VectorSubcoreMesh(core_axis_name="c", subcore_axis_name="s")

    @functools.partial(
        pl.kernel,
        mesh=mesh,
        compiler_params=pltpu.CompilerParams(needs_layout_passes=False),
        out_type=(
            jax.ShapeDtypeStruct((_NW, _NP), jnp.float32),
            jax.ShapeDtypeStruct((_NW, _NP), jnp.float32),
        ),
        scratch_types=[
            pltpu.VMEM((b_per_w,), jnp.float32),
            pltpu.VMEM((b_per_w,), jnp.int32),
            pltpu.VMEM((_NP,), jnp.float32),
            pltpu.VMEM((_NP,), jnp.float32),
        ],
    )
    def k(attn_hbm, dst_hbm, m_hbm, s_hbm, a_v, d_v, m_v, s_v):
        wid = lax.axis_index("s") * _NC + lax.axis_index("c")
        base = wid * b_per_w
        pltpu.sync_copy(attn_hbm.at[pl.ds(base, b_per_w)], a_v)
        pltpu.sync_copy(dst_hbm.at[pl.ds(base, b_per_w)], d_v)

        neg = jnp.full((_L,), -1e30, jnp.float32)
        zero = jnp.zeros((_L,), jnp.float32)

        def init_body(i, c):
            m_v[pl.ds(i * _L, _L)] = neg
            s_v[pl.ds(i * _L, _L)] = zero
            return c
        lax.fori_loop(0, _NP // _L, init_body, 0)

        def max_body(i, c):
            d = d_v[pl.ds(i * _L, _L)]
            a = a_v[pl.ds(i * _L, _L)]
            cur = plsc.load_gather(m_v, [d])
            plsc.store_scatter(m_v, [d], jnp.maximum(cur, a))
            return c
        lax.fori_loop(0, b_per_w // _L, max_body, 0)

        def sum_body(i, c):
            d = d_v[pl.ds(i * _L, _L)]
            a = a_v[pl.ds(i * _L, _L)]
            mv = plsc.load_gather(m_v, [d])
            plsc.addupdate_scatter(s_v, [d], jnp.exp(a - mv))
            return c
        lax.fori_loop(0, b_per_w // _L, sum_body, 0)

        pltpu.sync_copy(m_v, m_hbm.at[wid])
        pltpu.sync_copy(s_v, s_hbm.at[wid])

    return k(attn_p, dst_p)


def _merge_body(m_ref, s_ref, mg_ref, sg_ref):
    m = m_ref[...]
    s = s_ref[...]
    mg = jnp.max(m, axis=0, keepdims=True)
    scale = jnp.where(s > 0.0, jnp.exp(m - mg), 0.0)
    sg_ref[...] = jnp.sum(s * scale, axis=0, keepdims=True)
    mg_ref[...] = mg


def _merge_stats(m_parts, s_parts, block=2048):
    grid = (_NP // block,)
    return pl.pallas_call(
        _merge_body,
        grid=grid,
        in_specs=[
            pl.BlockSpec((_NW, block), lambda i: (0, i)),
            pl.BlockSpec((_NW, block), lambda i: (0, i)),
        ],
        out_specs=(
            pl.BlockSpec((1, block), lambda i: (0, i)),
            pl.BlockSpec((1, block), lambda i: (0, i)),
        ),
        out_shape=(
            jax.ShapeDtypeStruct((1, _NP), jnp.float32),
            jax.ShapeDtypeStruct((1, _NP), jnp.float32),
        ),
    )(m_parts, s_parts)


def _edge_weights(attn_p, dst_p, m_g, s_g):
    b_per_w = _EP // _NW
    mesh = plsc.VectorSubcoreMesh(core_axis_name="c", subcore_axis_name="s")

    @functools.partial(
        pl.kernel,
        mesh=mesh,
        compiler_params=pltpu.CompilerParams(needs_layout_passes=False),
        out_type=jax.ShapeDtypeStruct((_EP,), jnp.float32),
        scratch_types=[
            pltpu.VMEM((b_per_w,), jnp.float32),
            pltpu.VMEM((b_per_w,), jnp.int32),
            pltpu.VMEM((_NP,), jnp.float32),
            pltpu.VMEM((_NP,), jnp.float32),
        ],
    )
    def k(attn_hbm, dst_hbm, mg_hbm, sg_hbm, out_hbm, a_v, d_v, mg_v, sg_v):
        wid = lax.axis_index("s") * _NC + lax.axis_index("c")
        base = wid * b_per_w
        pltpu.sync_copy(attn_hbm.at[pl.ds(base, b_per_w)], a_v)
        pltpu.sync_copy(dst_hbm.at[pl.ds(base, b_per_w)], d_v)
        pltpu.sync_copy(mg_hbm.at[pl.ds(0, _NP)], mg_v)
        pltpu.sync_copy(sg_hbm.at[pl.ds(0, _NP)], sg_v)

        def w_body(i, c):
            d = d_v[pl.ds(i * _L, _L)]
            a = a_v[pl.ds(i * _L, _L)]
            mv = plsc.load_gather(mg_v, [d])
            sv = plsc.load_gather(sg_v, [d])
            a_v[pl.ds(i * _L, _L)] = jnp.exp(a - mv) / sv
            return c
        lax.fori_loop(0, b_per_w // _L, w_body, 0)

        pltpu.sync_copy(a_v, out_hbm.at[pl.ds(base, b_per_w)])

    return k(attn_p, dst_p, m_g, s_g)


def _scatter_rows(scaled, dst3, n_edges):
    e_per_tile = _EP // _NS
    n_chunks_full = e_per_tile // _CH
    rows_per_tile = _NP // _NS
    Dh = 128
    mesh = plsc.VectorSubcoreMesh(core_axis_name="c", subcore_axis_name="s")

    @functools.partial(
        pl.kernel,
        mesh=mesh,
        compiler_params=pltpu.CompilerParams(needs_layout_passes=False),
        out_type=jax.ShapeDtypeStruct((2, _NP, Dh), jnp.float32),
        scratch_types=[
            pltpu.VMEM((_CH,), jnp.int32),
            pltpu.VMEM((_CH, Dh), jnp.float32),
            pltpu.VMEM_SHARED((_NP, Dh), jnp.float32),
        ],
    )
    def k(scaled_hbm, dst3_hbm, nz_hbm, idx_v, rows_v, nz_sh):
        cid = lax.axis_index("c")
        sid = lax.axis_index("s")
        base = sid * e_per_tile

        zero = jnp.zeros((_L,), jnp.float32)

        def zr_body(r, c):
            for kk in range(Dh // _L):
                rows_v[r, pl.ds(kk * _L, _L)] = zero
            return c
        lax.fori_loop(0, _CH, zr_body, 0)
        for part in range(rows_per_tile // _CH):
            pltpu.sync_copy(
                rows_v,
                nz_sh.at[pl.ds(sid * rows_per_tile + part * _CH, _CH)])
        rem_rows = rows_per_tile % _CH
        if rem_rows:
            pltpu.sync_copy(
                rows_v.at[pl.ds(0, rem_rows)],
                nz_sh.at[pl.ds(sid * rows_per_tile
                               + (rows_per_tile // _CH) * _CH, rem_rows)])
        plsc.subcore_barrier()

        def chunk_body(j, c):
            off = base + j * _CH
            pltpu.sync_copy(scaled_hbm.at[cid, pl.ds(off, _CH)], rows_v)
            pltpu.sync_copy(dst3_hbm.at[sid, j], idx_v)
            pltpu.sync_copy(rows_v, nz_sh.at[idx_v], add=True)
            return c
        lax.fori_loop(0, n_chunks_full, chunk_body, 0)

        plsc.subcore_barrier()
        pltpu.sync_copy(
            nz_sh.at[pl.ds(sid * rows_per_tile, rows_per_tile)],
            nz_hbm.at[cid, pl.ds(sid * rows_per_tile, rows_per_tile)])

    return k(scaled, dst3)


def kernel(nh, eh, edge_index, Wn1, bn1, Wn2, bn2, We1, be1, We2, be2):
    N = nh.shape[0]
    src = edge_index[0]
    dst = edge_index[1]

    n_h = _mlp(nh, Wn1, bn1, Wn2, bn2, block_m=1000)
    e_h = _mlp(eh, We1, be1, We2, be2, block_m=2000)

    E = src.shape[0]
    src_nh, dst_nh = _multi_gather([(n_h, src), (n_h, dst)])
    attn = _attn_rows(src_nh, dst_nh, e_h).reshape(E)

    pad = _EP - E
    attn_p = jnp.concatenate([attn, jnp.full((pad,), -1e30, jnp.float32)])
    dst_p = jnp.concatenate([dst, jnp.full((pad,), N, jnp.int32)])
    dst3 = dst_p.reshape(_NS, (_EP // _NS) // _CH, _CH)

    m_parts, s_parts = _seg_stats(attn_p, dst_p)
    m_g, s_g = _merge_stats(m_parts, s_parts)
    attn_sm = _edge_weights(attn_p, dst_p,
                            m_g.reshape(-1), s_g.reshape(-1))[:E]

    scaled2 = jnp.concatenate(
        [_scale_rows_split(attn_sm, src_nh),
         jnp.zeros((2, _EP - E, 128), jnp.float32)], axis=1)
    nz3 = _scatter_rows(scaled2, dst3, E)
    nzl, nzr = nz3[0], nz3[1]
    n_out = _node_out(n_h, nzl[:N], nzr[:N])
    nsl, nsr, ndl, ndr = _multi_gather(
        [(nzl, src), (nzr, src), (nzl, dst), (nzr, dst)])
    e_out = _edge_out(e_h, nsl, nsr, ndl, ndr)
    return (n_out, e_out)
